# Initial kernel scaffold; baseline (speedup 1.0000x reference)
import jax
import jax.numpy as jnp
from jax.experimental import pallas as pl


def kernel(nodes, edges, receivers, senders, globals_, W, b):
    raise NotImplementedError("write your pallas kernel here")


# speedup vs baseline: 5.6209x; 5.6209x over previous
import functools

import jax
import jax.numpy as jnp
from jax import lax
from jax.experimental import pallas as pl
from jax.experimental.pallas import tpu as pltpu
from jax.experimental.pallas import tpu_sc as plsc

N_NODES = 100000
N_EDGES = 3200000
D_NODE = 128
D_EDGE = 16
D_GLOBAL = 16
D_OUT = 128

NC = 2
NS = 16
NW = NC * NS
G = 128
ROWS = N_EDGES // G
CHUNK = 8
VROWS = 4
NCHUNKS = ROWS // CHUNK
CH_PER_W = NCHUNKS // NW
CH_REM = NCHUNKS - CH_PER_W * NW
Z0 = 6256
Z_LAST = N_NODES - (NS - 1) * Z0


def _sc_body(edges_hbm, recv_hbm, zeros_hbm, out_hbm, agg, vbuf, ibuf):
    cid = lax.axis_index("c")
    sid = lax.axis_index("s")

    @pl.when(sid < NS - 1)
    def _():
        pltpu.sync_copy(zeros_hbm, agg.at[pl.ds(sid * Z0, Z0)])

    @pl.when(sid == NS - 1)
    def _():
        pltpu.sync_copy(zeros_hbm.at[pl.ds(0, Z_LAST)],
                        agg.at[pl.ds((NS - 1) * Z0, Z_LAST)])

    plsc.subcore_barrier()

    wid = cid * NS + sid
    base_chunk = wid * CH_PER_W + jnp.minimum(wid, CH_REM)
    nchunks = CH_PER_W + jnp.where(wid < CH_REM, 1, 0)

    def chunk_body(i, _):
        row0 = (base_chunk + i) * CHUNK
        pltpu.sync_copy(recv_hbm.at[pl.ds(row0, CHUNK)], ibuf)
        for h in range(CHUNK // VROWS):
            pltpu.sync_copy(
                edges_hbm.at[pl.ds((row0 + h * VROWS) * G, VROWS * G)], vbuf)
            for j in range(VROWS):
                pltpu.sync_copy(vbuf.at[pl.ds(j * G, G)],
                                agg.at[ibuf.at[h * VROWS + j]], add=True)
        return 0

    lax.fori_loop(0, nchunks, chunk_body, 0)

    plsc.subcore_barrier()

    @pl.when(sid < NS - 1)
    def _():
        pltpu.sync_copy(agg.at[pl.ds(sid * Z0, Z0)],
                        out_hbm.at[cid, pl.ds(sid * Z0, Z0)])

    @pl.when(sid == NS - 1)
    def _():
        pltpu.sync_copy(agg.at[pl.ds((NS - 1) * Z0, Z_LAST)],
                        out_hbm.at[cid, pl.ds((NS - 1) * Z0, Z_LAST)])


_seg_sum_sc = functools.partial(
    pl.kernel,
    out_type=jax.ShapeDtypeStruct((NC, N_NODES, D_EDGE), jnp.float32),
    mesh=plsc.VectorSubcoreMesh(core_axis_name="c", subcore_axis_name="s"),
    compiler_params=pltpu.CompilerParams(use_tc_tiling_on_sc=False),
    scratch_types=[
        pltpu.VMEM_SHARED((N_NODES, D_EDGE), jnp.float32),
        pltpu.VMEM((VROWS * G, D_EDGE), jnp.float32),
        pltpu.VMEM((CHUNK, G), jnp.int32),
    ],
)(_sc_body)


BLK = 2000


def _tc_body(nodes_ref, part_ref, w1_ref, w2_ref, g_ref, w3_ref, b_ref, out_ref):
    acc = jnp.dot(nodes_ref[...], w1_ref[...], preferred_element_type=jnp.float32)
    agg = part_ref[0, :, :] + part_ref[1, :, :]
    acc += jnp.dot(agg, w2_ref[...], preferred_element_type=jnp.float32)
    row = jnp.dot(g_ref[...], w3_ref[...], preferred_element_type=jnp.float32)
    out_ref[...] = jnp.maximum(acc + row + b_ref[...], 0.0)


def _fused_tc(nodes, partials, w1, w2, g, w3, b2):
    return pl.pallas_call(
        _tc_body,
        grid=(N_NODES // BLK,),
        in_specs=[
            pl.BlockSpec((BLK, D_NODE), lambda i: (i, 0)),
            pl.BlockSpec((NC, BLK, D_EDGE), lambda i: (0, i, 0)),
            pl.BlockSpec((D_NODE, D_OUT), lambda i: (0, 0)),
            pl.BlockSpec((D_EDGE, D_OUT), lambda i: (0, 0)),
            pl.BlockSpec((1, D_GLOBAL), lambda i: (0, 0)),
            pl.BlockSpec((D_GLOBAL, D_OUT), lambda i: (0, 0)),
            pl.BlockSpec((1, D_OUT), lambda i: (0, 0)),
        ],
        out_specs=pl.BlockSpec((BLK, D_OUT), lambda i: (i, 0)),
        out_shape=jax.ShapeDtypeStruct((N_NODES, D_OUT), jnp.float32),
    )(nodes, partials, w1, w2, g, w3, b2)


def kernel(nodes, edges, receivers, senders, globals_, W, b):
    recv2d = receivers.reshape(ROWS, G).astype(jnp.int32)
    zeros_init = jnp.zeros((Z0, D_EDGE), jnp.float32)
    partials = _seg_sum_sc(edges, recv2d, zeros_init)

    w1 = W[:D_NODE]
    w2 = W[D_NODE:D_NODE + D_EDGE]
    w3 = W[D_NODE + D_EDGE:]
    return _fused_tc(nodes, partials, w1, w2, globals_, w3, b.reshape(1, D_OUT))


# speedup vs baseline: 5.8416x; 1.0393x over previous
import functools

import jax
import jax.numpy as jnp
from jax import lax
from jax.experimental import pallas as pl
from jax.experimental.pallas import tpu as pltpu
from jax.experimental.pallas import tpu_sc as plsc

N_NODES = 100000
N_EDGES = 3200000
D_NODE = 128
D_EDGE = 16
D_GLOBAL = 16
D_OUT = 128

NC = 2
NS = 16
NW = NC * NS
G = 128
ROWS = N_EDGES // G
CHUNK = 8
VROWS = 4
NCHUNKS = ROWS // CHUNK
CH_PER_W = NCHUNKS // NW
CH_REM = NCHUNKS - CH_PER_W * NW
EPG = G * D_EDGE // 128
AGG_ROWS = N_NODES * D_EDGE // 128
Z0 = 6256
Z_LAST = N_NODES - (NS - 1) * Z0


def _sc_body(edges_hbm, recv_hbm, zeros_hbm, out_hbm, agg, vbuf, ibuf):
    cid = lax.axis_index("c")
    sid = lax.axis_index("s")

    @pl.when(sid < NS - 1)
    def _():
        pltpu.sync_copy(zeros_hbm, agg.at[pl.ds(sid * Z0, Z0)])

    @pl.when(sid == NS - 1)
    def _():
        pltpu.sync_copy(zeros_hbm.at[pl.ds(0, Z_LAST)],
                        agg.at[pl.ds((NS - 1) * Z0, Z_LAST)])

    plsc.subcore_barrier()

    wid = cid * NS + sid
    base_chunk = wid * CH_PER_W + jnp.minimum(wid, CH_REM)
    nchunks = CH_PER_W + jnp.where(wid < CH_REM, 1, 0)

    def chunk_body(i, _):
        row0 = (base_chunk + i) * CHUNK
        pltpu.sync_copy(recv_hbm.at[pl.ds(row0, CHUNK)], ibuf)
        for h in range(CHUNK // VROWS):
            g0 = row0 + h * VROWS
            pltpu.sync_copy(edges_hbm.at[pl.ds(g0 * G, VROWS * G)], vbuf)
            for j in range(VROWS):
                pltpu.sync_copy(vbuf.at[pl.ds(j * G, G)],
                                agg.at[ibuf.at[h * VROWS + j]], add=True)
        return 0

    lax.fori_loop(0, nchunks, chunk_body, 0)

    plsc.subcore_barrier()

    @pl.when(sid < NS - 1)
    def _():
        pltpu.sync_copy(agg.at[pl.ds(sid * Z0, Z0)],
                        out_hbm.at[cid, pl.ds(sid * Z0, Z0)])

    @pl.when(sid == NS - 1)
    def _():
        pltpu.sync_copy(agg.at[pl.ds((NS - 1) * Z0, Z_LAST)],
                        out_hbm.at[cid, pl.ds((NS - 1) * Z0, Z_LAST)])


_seg_sum_sc = functools.partial(
    pl.kernel,
    out_type=jax.ShapeDtypeStruct((NC, N_NODES, D_EDGE), jnp.float32),
    mesh=plsc.VectorSubcoreMesh(core_axis_name="c", subcore_axis_name="s"),
    compiler_params=pltpu.CompilerParams(use_tc_tiling_on_sc=False),
    scratch_types=[
        pltpu.VMEM_SHARED((N_NODES, D_EDGE), jnp.float32),
        pltpu.VMEM((VROWS * G, D_EDGE), jnp.float32),
        pltpu.VMEM((CHUNK, G), jnp.int32),
    ],
)(_sc_body)


BLK = 2048
PBLK = BLK * D_EDGE // 128


def _tc_body(nodes_ref, part_ref, w1_ref, bd_ref, g_ref, w3_ref, b_ref, out_ref):
    acc = jnp.dot(nodes_ref[...], w1_ref[...], preferred_element_type=jnp.float32)
    psum = part_ref[0, :, :] + part_ref[1, :, :]
    e = jnp.dot(psum, bd_ref[...], preferred_element_type=jnp.float32)
    acc += e.reshape(BLK, D_OUT)
    row = jnp.dot(g_ref[...], w3_ref[...], preferred_element_type=jnp.float32)
    out_ref[...] = jnp.maximum(acc + row + b_ref[...], 0.0)


def _fused_tc(nodes, partials, w1, bd, g, w3, b2):
    return pl.pallas_call(
        _tc_body,
        grid=((N_NODES + BLK - 1) // BLK,),
        in_specs=[
            pl.BlockSpec((BLK, D_NODE), lambda i: (i, 0)),
            pl.BlockSpec((NC, PBLK, 128), lambda i: (0, i, 0)),
            pl.BlockSpec((D_NODE, D_OUT), lambda i: (0, 0)),
            pl.BlockSpec((D_NODE, 8 * D_OUT), lambda i: (0, 0)),
            pl.BlockSpec((1, D_GLOBAL), lambda i: (0, 0)),
            pl.BlockSpec((D_GLOBAL, D_OUT), lambda i: (0, 0)),
            pl.BlockSpec((1, D_OUT), lambda i: (0, 0)),
        ],
        out_specs=pl.BlockSpec((BLK, D_OUT), lambda i: (i, 0)),
        out_shape=jax.ShapeDtypeStruct((N_NODES, D_OUT), jnp.float32),
    )(nodes, partials, w1, bd, g, w3, b2)


def kernel(nodes, edges, receivers, senders, globals_, W, b):
    recv2d = receivers.reshape(ROWS, G).astype(jnp.int32)
    zeros_init = jnp.zeros((Z0, D_EDGE), jnp.float32)
    edges128 = lax.optimization_barrier(edges.reshape(N_EDGES // 8, 128))
    edges_lin = edges128.reshape(N_EDGES, D_EDGE)
    partials = _seg_sum_sc(edges_lin, recv2d, zeros_init)
    pview = partials.reshape(NC, AGG_ROWS, 128)

    w1 = W[:D_NODE]
    w2 = W[D_NODE:D_NODE + D_EDGE]
    w3 = W[D_NODE + D_EDGE:]
    bd = jnp.kron(jnp.eye(8, dtype=jnp.float32), w2)
    return _fused_tc(nodes, pview, w1, bd, globals_, w3, b.reshape(1, D_OUT))


# speedup vs baseline: 10.2406x; 1.7531x over previous
import functools

import jax
import jax.numpy as jnp
from jax import lax
from jax.experimental import pallas as pl
from jax.experimental.pallas import tpu as pltpu
from jax.experimental.pallas import tpu_sc as plsc

N_NODES = 100000
N_EDGES = 3200000
D_NODE = 128
D_EDGE = 16
D_GLOBAL = 16
D_OUT = 128

NC = 2
NS = 16
NW = NC * NS
G = 128
ROWS = N_EDGES // G
CHUNK = 8
VROWS = 4
NCHUNKS = ROWS // CHUNK
CH_PER_W = NCHUNKS // NW
CH_REM = NCHUNKS - CH_PER_W * NW
EPG = G * D_EDGE // 128
AGG_ROWS = N_NODES * D_EDGE // 128
Z0 = 6256
Z_LAST = N_NODES - (NS - 1) * Z0


GS = 2


def _sc_body(edges_hbm, recv_hbm, zeros_hbm, out_hbm, agg, vbuff, vbuft, ibuf):
    cid = lax.axis_index("c")
    sid = lax.axis_index("s")
    iota = lax.iota(jnp.int32, 16)
    cols = [jnp.full((16,), d, jnp.int32) for d in range(D_EDGE)]

    @pl.when(sid < NS - 1)
    def _():
        pltpu.sync_copy(zeros_hbm, agg.at[pl.ds(sid * Z0, Z0)])

    @pl.when(sid == NS - 1)
    def _():
        pltpu.sync_copy(zeros_hbm.at[pl.ds(0, Z_LAST)],
                        agg.at[pl.ds((NS - 1) * Z0, Z_LAST)])

    plsc.subcore_barrier()

    wid = cid * NS + sid
    base_chunk = wid * CH_PER_W + jnp.minimum(wid, CH_REM)
    nchunks = CH_PER_W + jnp.where(wid < CH_REM, 1, 0)

    def chunk_body(i, _):
        row0 = (base_chunk + i) * CHUNK
        pltpu.sync_copy(recv_hbm.at[pl.ds(row0, CHUNK)], ibuf)
        for h in range(CHUNK // GS):
            g0 = row0 + h * GS
            pltpu.sync_copy(edges_hbm.at[:, pl.ds(g0, GS)], vbuff)
            for gs in range(GS):
                def tr_body(k, _, gs=gs):
                    rows = iota + (gs * G + k * 16)
                    for d in range(D_EDGE):
                        v = vbuff[d // 8, gs, d % 8, pl.ds(k * 16, 16)]
                        plsc.store_scatter(vbuft, [rows, cols[d]], v)
                    return 0

                lax.fori_loop(0, G // 16, tr_body, 0)
            for gs in range(GS):
                pltpu.sync_copy(vbuft.at[pl.ds(gs * G, G)],
                                agg.at[ibuf.at[h * GS + gs]], add=True)
        return 0

    lax.fori_loop(0, nchunks, chunk_body, 0)

    plsc.subcore_barrier()

    @pl.when(sid < NS - 1)
    def _():
        pltpu.sync_copy(agg.at[pl.ds(sid * Z0, Z0)],
                        out_hbm.at[cid, pl.ds(sid * Z0, Z0)])

    @pl.when(sid == NS - 1)
    def _():
        pltpu.sync_copy(agg.at[pl.ds((NS - 1) * Z0, Z_LAST)],
                        out_hbm.at[cid, pl.ds((NS - 1) * Z0, Z_LAST)])


_seg_sum_sc = functools.partial(
    pl.kernel,
    out_type=jax.ShapeDtypeStruct((NC, N_NODES, D_EDGE), jnp.float32),
    mesh=plsc.VectorSubcoreMesh(core_axis_name="c", subcore_axis_name="s"),
    compiler_params=pltpu.CompilerParams(use_tc_tiling_on_sc=False,
                                         needs_layout_passes=False),
    scratch_types=[
        pltpu.VMEM_SHARED((N_NODES, D_EDGE), jnp.float32),
        pltpu.VMEM((2, GS, 8, G), jnp.float32),
        pltpu.VMEM((GS * G, D_EDGE), jnp.float32),
        pltpu.VMEM((CHUNK, G), jnp.int32),
    ],
)(_sc_body)


BLK = 2048
PBLK = BLK * D_EDGE // 128


def _tc_body(nodes_ref, part_ref, w1_ref, bd_ref, g_ref, w3_ref, b_ref, out_ref):
    acc = jnp.dot(nodes_ref[...], w1_ref[...], preferred_element_type=jnp.float32)
    psum = part_ref[0, :, :] + part_ref[1, :, :]
    e = jnp.dot(psum, bd_ref[...], preferred_element_type=jnp.float32)
    acc += e.reshape(BLK, D_OUT)
    row = jnp.dot(g_ref[...], w3_ref[...], preferred_element_type=jnp.float32)
    out_ref[...] = jnp.maximum(acc + row + b_ref[...], 0.0)


def _fused_tc(nodes, partials, w1, bd, g, w3, b2):
    return pl.pallas_call(
        _tc_body,
        grid=((N_NODES + BLK - 1) // BLK,),
        in_specs=[
            pl.BlockSpec((BLK, D_NODE), lambda i: (i, 0)),
            pl.BlockSpec((NC, PBLK, 128), lambda i: (0, i, 0)),
            pl.BlockSpec((D_NODE, D_OUT), lambda i: (0, 0)),
            pl.BlockSpec((D_NODE, 8 * D_OUT), lambda i: (0, 0)),
            pl.BlockSpec((1, D_GLOBAL), lambda i: (0, 0)),
            pl.BlockSpec((D_GLOBAL, D_OUT), lambda i: (0, 0)),
            pl.BlockSpec((1, D_OUT), lambda i: (0, 0)),
        ],
        out_specs=pl.BlockSpec((BLK, D_OUT), lambda i: (i, 0)),
        out_shape=jax.ShapeDtypeStruct((N_NODES, D_OUT), jnp.float32),
    )(nodes, partials, w1, bd, g, w3, b2)


def kernel(nodes, edges, receivers, senders, globals_, W, b):
    recv2d = receivers.reshape(ROWS, G).astype(jnp.int32)
    zeros_init = jnp.zeros((Z0, D_EDGE), jnp.float32)
    e4 = jnp.transpose(edges.reshape(ROWS, G, 2, 8), (2, 0, 3, 1))
    partials = _seg_sum_sc(e4, recv2d, zeros_init)
    pview = partials.reshape(NC, AGG_ROWS, 128)

    w1 = W[:D_NODE]
    w2 = W[D_NODE:D_NODE + D_EDGE]
    w3 = W[D_NODE + D_EDGE:]
    bd = jnp.kron(jnp.eye(8, dtype=jnp.float32), w2)
    return _fused_tc(nodes, pview, w1, bd, globals_, w3, b.reshape(1, D_OUT))


# speedup vs baseline: 12.7637x; 1.2464x over previous
import functools

import jax
import jax.numpy as jnp
from jax import lax
from jax.experimental import pallas as pl
from jax.experimental.pallas import tpu as pltpu
from jax.experimental.pallas import tpu_sc as plsc

N_NODES = 100000
N_EDGES = 3200000
D_NODE = 128
D_EDGE = 16
D_GLOBAL = 16
D_OUT = 128

NC = 2
NS = 16
NW = NC * NS
G = 128
ROWS = N_EDGES // G
CHUNK = 8
VROWS = 4
NCHUNKS = ROWS // CHUNK
CH_PER_W = NCHUNKS // NW
CH_REM = NCHUNKS - CH_PER_W * NW
EPG = G * D_EDGE // 128
AGG_ROWS = N_NODES * D_EDGE // 128
Z0 = 6256
Z_LAST = N_NODES - (NS - 1) * Z0


def _sc_body(edges_hbm, recv_hbm, zeros_hbm, out_hbm, agg,
             f0, f1, t0, t1, ibuf, fs0, fs1, ss0, ss1):
    cid = lax.axis_index("c")
    sid = lax.axis_index("s")
    iota = lax.iota(jnp.int32, 16)
    cols = [jnp.full((16,), d, jnp.int32) for d in range(D_EDGE)]
    fbuf = (f0, f1)
    tbuf = (t0, t1)
    fsem = (fs0, fs1)
    ssem = (ss0, ss1)

    @pl.when(sid < NS - 1)
    def _():
        pltpu.sync_copy(zeros_hbm, agg.at[pl.ds(sid * Z0, Z0)])

    @pl.when(sid == NS - 1)
    def _():
        pltpu.sync_copy(zeros_hbm.at[pl.ds(0, Z_LAST)],
                        agg.at[pl.ds((NS - 1) * Z0, Z_LAST)])

    plsc.subcore_barrier()

    wid = cid * NS + sid
    base_chunk = wid * CH_PER_W + jnp.minimum(wid, CH_REM)
    nchunks = CH_PER_W + jnp.where(wid < CH_REM, 1, 0)

    def fcopy(g, b):
        return pltpu.make_async_copy(edges_hbm.at[:, pl.ds(g, 1)],
                                     fbuf[b], fsem[b])

    def sdrain(b, hrow):
        pltpu.make_async_copy(tbuf[b], agg.at[ibuf.at[hrow]], ssem[b]).wait()

    def chunk_body(i, _):
        row0 = (base_chunk + i) * CHUNK
        pltpu.sync_copy(recv_hbm.at[pl.ds(row0, CHUNK)], ibuf)
        fcopy(row0, 0).start()
        for h in range(CHUNK):
            b = h % 2
            fcopy(row0 + h, b).wait()
            if h + 1 < CHUNK:
                fcopy(row0 + h + 1, 1 - b).start()
            if h >= 2:
                sdrain(b, h - 2)
            def tr_body(k, _, b=b):
                rows = iota + k * 16
                for d in range(D_EDGE):
                    v = fbuf[b][d // 8, 0, d % 8, pl.ds(k * 16, 16)]
                    plsc.store_scatter(tbuf[b], [rows, cols[d]], v)
                return 0

            lax.fori_loop(0, G // 16, tr_body, 0)
            pltpu.async_copy(tbuf[b], agg.at[ibuf.at[h]], ssem[b], add=True)
        sdrain(0, CHUNK - 2)
        sdrain(1, CHUNK - 1)
        return 0

    lax.fori_loop(0, nchunks, chunk_body, 0)

    plsc.subcore_barrier()

    @pl.when(sid < NS - 1)
    def _():
        pltpu.sync_copy(agg.at[pl.ds(sid * Z0, Z0)],
                        out_hbm.at[cid, pl.ds(sid * Z0, Z0)])

    @pl.when(sid == NS - 1)
    def _():
        pltpu.sync_copy(agg.at[pl.ds((NS - 1) * Z0, Z_LAST)],
                        out_hbm.at[cid, pl.ds((NS - 1) * Z0, Z_LAST)])


_seg_sum_sc = functools.partial(
    pl.kernel,
    out_type=jax.ShapeDtypeStruct((NC, N_NODES, D_EDGE), jnp.float32),
    mesh=plsc.VectorSubcoreMesh(core_axis_name="c", subcore_axis_name="s"),
    compiler_params=pltpu.CompilerParams(use_tc_tiling_on_sc=False,
                                         needs_layout_passes=False),
    scratch_types=[
        pltpu.VMEM_SHARED((N_NODES, D_EDGE), jnp.float32),
        pltpu.VMEM((2, 1, 8, G), jnp.float32),
        pltpu.VMEM((2, 1, 8, G), jnp.float32),
        pltpu.VMEM((G, D_EDGE), jnp.float32),
        pltpu.VMEM((G, D_EDGE), jnp.float32),
        pltpu.VMEM((CHUNK, G), jnp.int32),
        pltpu.SemaphoreType.DMA,
        pltpu.SemaphoreType.DMA,
        pltpu.SemaphoreType.DMA,
        pltpu.SemaphoreType.DMA,
    ],
)(_sc_body)


BLK = 2048
PBLK = BLK * D_EDGE // 128


def _tc_body(nodes_ref, part_ref, w1_ref, bd_ref, g_ref, w3_ref, b_ref, out_ref):
    acc = jnp.dot(nodes_ref[...], w1_ref[...], preferred_element_type=jnp.float32)
    psum = part_ref[0, :, :] + part_ref[1, :, :]
    e = jnp.dot(psum, bd_ref[...], preferred_element_type=jnp.float32)
    acc += e.reshape(BLK, D_OUT)
    row = jnp.dot(g_ref[...], w3_ref[...], preferred_element_type=jnp.float32)
    out_ref[...] = jnp.maximum(acc + row + b_ref[...], 0.0)


def _fused_tc(nodes, partials, w1, bd, g, w3, b2):
    return pl.pallas_call(
        _tc_body,
        grid=((N_NODES + BLK - 1) // BLK,),
        in_specs=[
            pl.BlockSpec((BLK, D_NODE), lambda i: (i, 0)),
            pl.BlockSpec((NC, PBLK, 128), lambda i: (0, i, 0)),
            pl.BlockSpec((D_NODE, D_OUT), lambda i: (0, 0)),
            pl.BlockSpec((D_NODE, 8 * D_OUT), lambda i: (0, 0)),
            pl.BlockSpec((1, D_GLOBAL), lambda i: (0, 0)),
            pl.BlockSpec((D_GLOBAL, D_OUT), lambda i: (0, 0)),
            pl.BlockSpec((1, D_OUT), lambda i: (0, 0)),
        ],
        out_specs=pl.BlockSpec((BLK, D_OUT), lambda i: (i, 0)),
        out_shape=jax.ShapeDtypeStruct((N_NODES, D_OUT), jnp.float32),
    )(nodes, partials, w1, bd, g, w3, b2)


def kernel(nodes, edges, receivers, senders, globals_, W, b):
    recv2d = receivers.reshape(ROWS, G).astype(jnp.int32)
    zeros_init = jnp.zeros((Z0, D_EDGE), jnp.float32)
    e4 = jnp.transpose(edges.reshape(ROWS, G, 2, 8), (2, 0, 3, 1))
    partials = _seg_sum_sc(e4, recv2d, zeros_init)
    pview = partials.reshape(NC, AGG_ROWS, 128)

    w1 = W[:D_NODE]
    w2 = W[D_NODE:D_NODE + D_EDGE]
    w3 = W[D_NODE + D_EDGE:]
    bd = jnp.kron(jnp.eye(8, dtype=jnp.float32), w2)
    return _fused_tc(nodes, pview, w1, bd, globals_, w3, b.reshape(1, D_OUT))


# speedup vs baseline: 12.7842x; 1.0016x over previous
import functools

import jax
import jax.numpy as jnp
from jax import lax
from jax.experimental import pallas as pl
from jax.experimental.pallas import tpu as pltpu
from jax.experimental.pallas import tpu_sc as plsc

N_NODES = 100000
N_EDGES = 3200000
D_NODE = 128
D_EDGE = 16
D_GLOBAL = 16
D_OUT = 128

NC = 2
NS = 16
NW = NC * NS
G = 128
ROWS = N_EDGES // G
CHUNK = 8
VROWS = 4
NCHUNKS = ROWS // CHUNK
CH_PER_W = NCHUNKS // NW
CH_REM = NCHUNKS - CH_PER_W * NW
EPG = G * D_EDGE // 128
AGG_ROWS = N_NODES * D_EDGE // 128
Z0 = 6256
Z_LAST = N_NODES - (NS - 1) * Z0


def _sc_body(edges_hbm, recv_hbm, zeros_hbm, out_hbm, agg,
             f0, f1, t0, t1, t2, t3, ibuf, fs0, fs1, ss0, ss1, ss2, ss3):
    cid = lax.axis_index("c")
    sid = lax.axis_index("s")
    iota = lax.iota(jnp.int32, 16)
    cols = [jnp.full((16,), d, jnp.int32) for d in range(D_EDGE)]
    fbuf = (f0, f1)
    tbuf = (t0, t1, t2, t3)
    fsem = (fs0, fs1)
    ssem = (ss0, ss1, ss2, ss3)

    @pl.when(sid < NS - 1)
    def _():
        pltpu.sync_copy(zeros_hbm, agg.at[pl.ds(sid * Z0, Z0)])

    @pl.when(sid == NS - 1)
    def _():
        pltpu.sync_copy(zeros_hbm.at[pl.ds(0, Z_LAST)],
                        agg.at[pl.ds((NS - 1) * Z0, Z_LAST)])

    plsc.subcore_barrier()

    wid = cid * NS + sid
    base_chunk = wid * CH_PER_W + jnp.minimum(wid, CH_REM)
    nchunks = CH_PER_W + jnp.where(wid < CH_REM, 1, 0)

    def fcopy(g, b):
        return pltpu.make_async_copy(edges_hbm.at[:, pl.ds(g, 1)],
                                     fbuf[b], fsem[b])

    def sdrain(b, hrow):
        pltpu.make_async_copy(tbuf[b], agg.at[ibuf.at[hrow]], ssem[b]).wait()

    def chunk_body(i, _):
        row0 = (base_chunk + i) * CHUNK
        pltpu.sync_copy(recv_hbm.at[pl.ds(row0, CHUNK)], ibuf)
        fcopy(row0, 0).start()
        for h in range(CHUNK):
            b = h % 2
            s = h % 4
            fcopy(row0 + h, b).wait()
            if h + 1 < CHUNK:
                fcopy(row0 + h + 1, 1 - b).start()
            if h >= 4:
                sdrain(s, h - 4)
            def tr_body(k, _, b=b, s=s):
                rows = iota + k * 16
                for d in range(D_EDGE):
                    v = fbuf[b][d // 8, 0, d % 8, pl.ds(k * 16, 16)]
                    plsc.store_scatter(tbuf[s], [rows, cols[d]], v)
                return 0

            lax.fori_loop(0, G // 16, tr_body, 0, unroll=2)
            pltpu.async_copy(tbuf[s], agg.at[ibuf.at[h]], ssem[s], add=True)
        for s in range(4):
            sdrain(s, CHUNK - 4 + s)
        return 0

    lax.fori_loop(0, nchunks, chunk_body, 0)

    plsc.subcore_barrier()

    @pl.when(sid < NS - 1)
    def _():
        pltpu.sync_copy(agg.at[pl.ds(sid * Z0, Z0)],
                        out_hbm.at[cid, pl.ds(sid * Z0, Z0)])

    @pl.when(sid == NS - 1)
    def _():
        pltpu.sync_copy(agg.at[pl.ds((NS - 1) * Z0, Z_LAST)],
                        out_hbm.at[cid, pl.ds((NS - 1) * Z0, Z_LAST)])


_seg_sum_sc = functools.partial(
    pl.kernel,
    out_type=jax.ShapeDtypeStruct((NC, N_NODES, D_EDGE), jnp.float32),
    mesh=plsc.VectorSubcoreMesh(core_axis_name="c", subcore_axis_name="s"),
    compiler_params=pltpu.CompilerParams(use_tc_tiling_on_sc=False,
                                         needs_layout_passes=False),
    scratch_types=[
        pltpu.VMEM_SHARED((N_NODES, D_EDGE), jnp.float32),
        pltpu.VMEM((2, 1, 8, G), jnp.float32),
        pltpu.VMEM((2, 1, 8, G), jnp.float32),
        pltpu.VMEM((G, D_EDGE), jnp.float32),
        pltpu.VMEM((G, D_EDGE), jnp.float32),
        pltpu.VMEM((G, D_EDGE), jnp.float32),
        pltpu.VMEM((G, D_EDGE), jnp.float32),
        pltpu.VMEM((CHUNK, G), jnp.int32),
        pltpu.SemaphoreType.DMA,
        pltpu.SemaphoreType.DMA,
        pltpu.SemaphoreType.DMA,
        pltpu.SemaphoreType.DMA,
        pltpu.SemaphoreType.DMA,
        pltpu.SemaphoreType.DMA,
    ],
)(_sc_body)


BLK = 2048
PBLK = BLK * D_EDGE // 128


def _tc_body(nodes_ref, part_ref, w1_ref, bd_ref, g_ref, w3_ref, b_ref, out_ref):
    acc = jnp.dot(nodes_ref[...], w1_ref[...], preferred_element_type=jnp.float32)
    psum = part_ref[0, :, :] + part_ref[1, :, :]
    e = jnp.dot(psum, bd_ref[...], preferred_element_type=jnp.float32)
    acc += e.reshape(BLK, D_OUT)
    row = jnp.dot(g_ref[...], w3_ref[...], preferred_element_type=jnp.float32)
    out_ref[...] = jnp.maximum(acc + row + b_ref[...], 0.0)


def _fused_tc(nodes, partials, w1, bd, g, w3, b2):
    return pl.pallas_call(
        _tc_body,
        grid=((N_NODES + BLK - 1) // BLK,),
        in_specs=[
            pl.BlockSpec((BLK, D_NODE), lambda i: (i, 0)),
            pl.BlockSpec((NC, PBLK, 128), lambda i: (0, i, 0)),
            pl.BlockSpec((D_NODE, D_OUT), lambda i: (0, 0)),
            pl.BlockSpec((D_NODE, 8 * D_OUT), lambda i: (0, 0)),
            pl.BlockSpec((1, D_GLOBAL), lambda i: (0, 0)),
            pl.BlockSpec((D_GLOBAL, D_OUT), lambda i: (0, 0)),
            pl.BlockSpec((1, D_OUT), lambda i: (0, 0)),
        ],
        out_specs=pl.BlockSpec((BLK, D_OUT), lambda i: (i, 0)),
        out_shape=jax.ShapeDtypeStruct((N_NODES, D_OUT), jnp.float32),
    )(nodes, partials, w1, bd, g, w3, b2)


def kernel(nodes, edges, receivers, senders, globals_, W, b):
    recv2d = receivers.reshape(ROWS, G).astype(jnp.int32)
    zeros_init = jnp.zeros((Z0, D_EDGE), jnp.float32)
    e4 = jnp.transpose(edges.reshape(ROWS, G, 2, 8), (2, 0, 3, 1))
    partials = _seg_sum_sc(e4, recv2d, zeros_init)
    pview = partials.reshape(NC, AGG_ROWS, 128)

    w1 = W[:D_NODE]
    w2 = W[D_NODE:D_NODE + D_EDGE]
    w3 = W[D_NODE + D_EDGE:]
    bd = jnp.kron(jnp.eye(8, dtype=jnp.float32), w2)
    return _fused_tc(nodes, pview, w1, bd, globals_, w3, b.reshape(1, D_OUT))


# speedup vs baseline: 17.7384x; 1.3875x over previous
import functools

import jax
import jax.numpy as jnp
from jax import lax
from jax.experimental import pallas as pl
from jax.experimental.pallas import tpu as pltpu
from jax.experimental.pallas import tpu_sc as plsc

N_NODES = 100000
N_EDGES = 3200000
D_NODE = 128
D_EDGE = 16
D_GLOBAL = 16
D_OUT = 128

NC = 2
NS = 16
NW = NC * NS
G = 128
ROWS = N_EDGES // G
CHUNK = 8
VROWS = 4
NCHUNKS = ROWS // CHUNK
CH_PER_W = NCHUNKS // NW
CH_REM = NCHUNKS - CH_PER_W * NW
EPG = G * D_EDGE // 128
AGG_ROWS = N_NODES * D_EDGE // 128
Z0 = 6256
Z_LAST = N_NODES - (NS - 1) * Z0


def _sc_body(edges_hbm, recv_hbm, zeros_hbm, out_hbm, agg,
             f0, f1, f2, f3, t0, t1, ibuf, fs0, fs1, fs2, fs3, ss0, ss1):
    cid = lax.axis_index("c")
    sid = lax.axis_index("s")
    iota = lax.iota(jnp.int32, 16)
    cols = [jnp.full((16,), d, jnp.int32) for d in range(D_EDGE)]
    fbuf = (f0, f1, f2, f3)
    tbuf = (t0, t1)
    fsem = (fs0, fs1, fs2, fs3)
    ssem = (ss0, ss1)

    @pl.when(sid < NS - 1)
    def _():
        pltpu.sync_copy(zeros_hbm, agg.at[pl.ds(sid * Z0, Z0)])

    @pl.when(sid == NS - 1)
    def _():
        pltpu.sync_copy(zeros_hbm.at[pl.ds(0, Z_LAST)],
                        agg.at[pl.ds((NS - 1) * Z0, Z_LAST)])

    plsc.subcore_barrier()

    wid = cid * NS + sid
    base_chunk = wid * CH_PER_W + jnp.minimum(wid, CH_REM)
    nchunks = CH_PER_W + jnp.where(wid < CH_REM, 1, 0)
    end_row = (base_chunk + nchunks) * CHUNK

    def fcopy(g, b):
        return pltpu.make_async_copy(edges_hbm.at[:, pl.ds(g, 1)],
                                     fbuf[b], fsem[b])

    def sdrain(b, hrow):
        pltpu.make_async_copy(tbuf[b], agg.at[ibuf.at[hrow]], ssem[b]).wait()

    for p in range(3):
        fcopy(base_chunk * CHUNK + p, p).start()

    def chunk_body(i, _):
        row0 = (base_chunk + i) * CHUNK
        pltpu.sync_copy(recv_hbm.at[pl.ds(row0, CHUNK)], ibuf)
        for h in range(CHUNK):
            b = h % 4
            s = h % 2
            fcopy(row0 + h, b).wait()
            nxt = row0 + h + 3

            @pl.when(nxt < end_row)
            def _(b=b, nxt=nxt):
                fcopy(nxt, (b + 3) % 4).start()

            if h >= 2:
                sdrain(s, h - 2)
            def tr_body(k, _, b=b, s=s):
                rows = iota + k * 16
                for d in range(D_EDGE):
                    v = fbuf[b][d // 8, 0, d % 8, pl.ds(k * 16, 16)]
                    plsc.store_scatter(tbuf[s], [rows, cols[d]], v)
                return 0

            lax.fori_loop(0, G // 16, tr_body, 0, unroll=2)
            pltpu.async_copy(tbuf[s], agg.at[ibuf.at[h]], ssem[s], add=True)
        sdrain(0, CHUNK - 2)
        sdrain(1, CHUNK - 1)
        return 0

    lax.fori_loop(0, nchunks, chunk_body, 0)

    plsc.subcore_barrier()

    @pl.when(sid < NS - 1)
    def _():
        pltpu.sync_copy(agg.at[pl.ds(sid * Z0, Z0)],
                        out_hbm.at[cid, pl.ds(sid * Z0, Z0)])

    @pl.when(sid == NS - 1)
    def _():
        pltpu.sync_copy(agg.at[pl.ds((NS - 1) * Z0, Z_LAST)],
                        out_hbm.at[cid, pl.ds((NS - 1) * Z0, Z_LAST)])


_seg_sum_sc = functools.partial(
    pl.kernel,
    out_type=jax.ShapeDtypeStruct((NC, N_NODES, D_EDGE), jnp.float32),
    mesh=plsc.VectorSubcoreMesh(core_axis_name="c", subcore_axis_name="s"),
    compiler_params=pltpu.CompilerParams(use_tc_tiling_on_sc=False,
                                         needs_layout_passes=False),
    scratch_types=[
        pltpu.VMEM_SHARED((N_NODES, D_EDGE), jnp.float32),
        pltpu.VMEM((2, 1, 8, G), jnp.float32),
        pltpu.VMEM((2, 1, 8, G), jnp.float32),
        pltpu.VMEM((2, 1, 8, G), jnp.float32),
        pltpu.VMEM((2, 1, 8, G), jnp.float32),
        pltpu.VMEM((G, D_EDGE), jnp.float32),
        pltpu.VMEM((G, D_EDGE), jnp.float32),
        pltpu.VMEM((CHUNK, G), jnp.int32),
        pltpu.SemaphoreType.DMA,
        pltpu.SemaphoreType.DMA,
        pltpu.SemaphoreType.DMA,
        pltpu.SemaphoreType.DMA,
        pltpu.SemaphoreType.DMA,
        pltpu.SemaphoreType.DMA,
    ],
)(_sc_body)


BLK = 2048
PBLK = BLK * D_EDGE // 128


def _tc_body(nodes_ref, part_ref, w1_ref, bd_ref, g_ref, w3_ref, b_ref, out_ref):
    acc = jnp.dot(nodes_ref[...], w1_ref[...], preferred_element_type=jnp.float32)
    psum = part_ref[0, :, :] + part_ref[1, :, :]
    e = jnp.dot(psum, bd_ref[...], preferred_element_type=jnp.float32)
    acc += e.reshape(BLK, D_OUT)
    row = jnp.dot(g_ref[...], w3_ref[...], preferred_element_type=jnp.float32)
    out_ref[...] = jnp.maximum(acc + row + b_ref[...], 0.0)


def _fused_tc(nodes, partials, w1, bd, g, w3, b2):
    return pl.pallas_call(
        _tc_body,
        grid=((N_NODES + BLK - 1) // BLK,),
        in_specs=[
            pl.BlockSpec((BLK, D_NODE), lambda i: (i, 0)),
            pl.BlockSpec((NC, PBLK, 128), lambda i: (0, i, 0)),
            pl.BlockSpec((D_NODE, D_OUT), lambda i: (0, 0)),
            pl.BlockSpec((D_NODE, 8 * D_OUT), lambda i: (0, 0)),
            pl.BlockSpec((1, D_GLOBAL), lambda i: (0, 0)),
            pl.BlockSpec((D_GLOBAL, D_OUT), lambda i: (0, 0)),
            pl.BlockSpec((1, D_OUT), lambda i: (0, 0)),
        ],
        out_specs=pl.BlockSpec((BLK, D_OUT), lambda i: (i, 0)),
        out_shape=jax.ShapeDtypeStruct((N_NODES, D_OUT), jnp.float32),
    )(nodes, partials, w1, bd, g, w3, b2)


def kernel(nodes, edges, receivers, senders, globals_, W, b):
    recv2d = receivers.reshape(ROWS, G).astype(jnp.int32)
    zeros_init = jnp.zeros((Z0, D_EDGE), jnp.float32)
    e4 = jnp.transpose(edges.reshape(ROWS, G, 2, 8), (2, 0, 3, 1))
    partials = _seg_sum_sc(e4, recv2d, zeros_init)
    pview = partials.reshape(NC, AGG_ROWS, 128)

    w1 = W[:D_NODE]
    w2 = W[D_NODE:D_NODE + D_EDGE]
    w3 = W[D_NODE + D_EDGE:]
    bd = jnp.kron(jnp.eye(8, dtype=jnp.float32), w2)
    return _fused_tc(nodes, pview, w1, bd, globals_, w3, b.reshape(1, D_OUT))
